# baseline (speedup 1.0000x reference)
import functools

import jax
import jax.numpy as jnp
from jax import lax
from jax.experimental import pallas as pl
from jax.experimental.pallas import tpu as pltpu
from jax.experimental.pallas import tpu_sc as plsc

_NC = 2
_NS = 16
_NW = _NC * _NS

_G = 128
_CHUNK = 512
_GPC = _CHUNK // _G


def _w0_repack_sc(w0):
    v, d = w0.shape
    wt = w0.T
    n_ch = v // _G
    v2 = n_ch * _G
    tail = w0[v2:, :].reshape((v - v2) // 2, 2 * d)
    n_it = (n_ch + _NW - 1) // _NW
    n_it2 = (n_it + 1) // 2

    mesh = plsc.VectorSubcoreMesh(core_axis_name="c", subcore_axis_name="s")

    @functools.partial(
        pl.kernel,
        out_type=jax.ShapeDtypeStruct((v // 2, 2 * d), jnp.float32),
        mesh=mesh,
        compiler_params=pltpu.CompilerParams(
            use_tc_tiling_on_sc=True, needs_layout_passes=False),
        scratch_types=[
            pltpu.VMEM((d, _G), jnp.float32),
            pltpu.VMEM((d, _G), jnp.float32),
            pltpu.VMEM((_G // 2, 2 * d), jnp.float32),
            pltpu.VMEM((_G // 2, 2 * d), jnp.float32),
            pltpu.SemaphoreType.DMA,
            pltpu.SemaphoreType.DMA,
            pltpu.SemaphoreType.DMA,
            pltpu.SemaphoreType.DMA,
        ],
    )
    def repack(wt_hbm, tail_hbm, w2_hbm, in0, in1, out0, out1,
               gs0, gs1, ws0, ws1):
        wid = lax.axis_index("s") * _NC + lax.axis_index("c")
        iota = lax.iota(jnp.int32, 16)
        inb = (in0, in1)
        outb = (out0, out1)
        gs = (gs0, gs1)
        ws = (ws0, ws1)

        @pl.when(wid == 0)
        def _():
            pltpu.sync_copy(tail_hbm, w2_hbm.at[pl.ds(v2 // 2, (v - v2) // 2)])

        def chunk_of(it):
            return it * _NW + wid

        def stage(it, bi):
            c = chunk_of(it)

            @pl.when(c < n_ch)
            def _():
                cb = c * _G
                for tr in range(d // 8):
                    pltpu.async_copy(
                        wt_hbm.at[pl.ds(tr * 8, 8), pl.ds(cb, _G)],
                        inb[bi].at[pl.ds(tr * 8, 8), :],
                        gs[bi],
                    )

        stage(0, 0)

        def body2(it2, carry):
            for b in range(2):
                it = it2 * 2 + b
                c = chunk_of(it)
                stage(it + 1, b ^ 1)

                @pl.when(c < n_ch)
                def _():
                    pltpu.make_async_copy(
                        wt_hbm.at[pl.ds(0, d), pl.ds(0, _G)], inb[b], gs[b]
                    ).wait()

                    @pl.when(it >= 2)
                    def _():
                        pltpu.make_async_copy(
                            w2_hbm.at[pl.ds(0, _G // 2)], outb[b], ws[b]
                        ).wait()

                    def trans_row(p, carry2):
                        col0 = 2 * p
                        for e in range(2):
                            cvec = jnp.full((16,), col0 + e, jnp.int32)
                            for j in range(d // 16):
                                g = plsc.load_gather(
                                    inb[b], [iota + j * 16, cvec]
                                )
                                outb[b][p, pl.ds(e * d + j * 16, 16)] = g
                        return carry2

                    lax.fori_loop(0, _G // 2, trans_row, 0)

                    pltpu.async_copy(
                        outb[b], w2_hbm.at[pl.ds(c * (_G // 2), _G // 2)], ws[b]
                    )
            return carry

        lax.fori_loop(0, n_it2, body2, 0)

        for b in range(2):
            pltpu.make_async_copy(
                w2_hbm.at[pl.ds(0, _G // 2)], outb[b], ws[b]
            ).wait()

    return repack(wt, tail)


def _x_gather_sc(x_flat, w0):
    n = x_flat.shape[0]
    d = w0.shape[1]
    per_w = n // _NW
    k = per_w // _G
    n_chunks = per_w // _CHUNK

    x3 = x_flat.reshape(_NW, k, _G)

    mesh = plsc.VectorSubcoreMesh(core_axis_name="c", subcore_axis_name="s")

    @functools.partial(
        pl.kernel,
        out_type=jax.ShapeDtypeStruct((n, d), jnp.float32),
        mesh=mesh,
        compiler_params=pltpu.CompilerParams(use_tc_tiling_on_sc=False),
        scratch_types=[
            pltpu.VMEM((k, _G), jnp.int32),
            pltpu.VMEM((_CHUNK, d), jnp.float32),
            pltpu.SemaphoreType.DMA,
        ],
    )
    def gather_kernel(x_hbm, w0_hbm, out_hbm, idx_v, rows_v, sem):
        wid = lax.axis_index("s") * _NC + lax.axis_index("c")
        base = wid * per_w
        pltpu.sync_copy(x_hbm.at[wid], idx_v)

        def chunk_body(c, carry):
            copies = []
            for g in range(_GPC):
                copies.append(
                    pltpu.async_copy(
                        w0_hbm.at[idx_v.at[c * _GPC + g]],
                        rows_v.at[pl.ds(g * _G, _G)],
                        sem,
                    )
                )
            for cp in copies:
                cp.wait()
            pltpu.sync_copy(rows_v, out_hbm.at[pl.ds(base + c * _CHUNK, _CHUNK)])
            return carry

        lax.fori_loop(0, n_chunks, chunk_body, 0)

    return gather_kernel(x3, w0)


def _y_embed_tc(y, w1):
    s, b = y.shape[1], y.shape[0]
    v, d = w1.shape
    rows = 2048
    nb = b // rows
    yt4 = y.T.reshape(s, nb, 1, rows)
    w1p = jnp.pad(w1, ((0, 16 - v), (0, 0)))

    def body(y_ref, w1_ref, o_ref):
        idx = y_ref[0, 0, 0, :]
        oh = (idx[:, None] == lax.broadcasted_iota(jnp.int32, (rows, 16), 1))
        o_ref[0] = jax.lax.dot(
            oh.astype(jnp.float32), w1_ref[...],
            precision=jax.lax.Precision.HIGHEST,
            preferred_element_type=jnp.float32,
        )

    out = pl.pallas_call(
        body,
        grid=(s, nb),
        in_specs=[
            pl.BlockSpec((1, 1, 1, rows), lambda i, j: (i, j, 0, 0)),
            pl.BlockSpec((16, d), lambda i, j: (0, 0)),
        ],
        out_specs=pl.BlockSpec((1, rows, d), lambda i, j: (i, j, 0)),
        out_shape=jax.ShapeDtypeStruct((s, b, d), jnp.float32),
    )(yt4, w1p)
    return out.transpose(1, 0, 2)


def kernel(x, w0, y, w1):
    b, s = x.shape
    n = b * s
    y_emb = _y_embed_tc(y.astype(jnp.int32), w1)
    w2 = _w0_repack_sc(w0)
    x_emb = _x_gather_sc(x.reshape(n).astype(jnp.int32),
                         w2.reshape(w0.shape[0], w0.shape[1]))
    return (x_emb.reshape(b, s, w0.shape[1]), y_emb)


# speedup vs baseline: 1.7821x; 1.7821x over previous
import functools

import jax
import jax.numpy as jnp
from jax import lax
from jax.experimental import pallas as pl
from jax.experimental.pallas import tpu as pltpu
from jax.experimental.pallas import tpu_sc as plsc

_NC = 2
_NS = 16
_NW = _NC * _NS

_G = 128
_CHUNK = 512
_GPC = _CHUNK // _G


def _x_gather_sc(x_flat, wpad, d):
    n = x_flat.shape[0]
    per_w = n // _NW
    k = per_w // _G
    n_chunks = per_w // _CHUNK

    x3 = x_flat.reshape(_NW, k, _G)

    mesh = plsc.VectorSubcoreMesh(core_axis_name="c", subcore_axis_name="s")

    @functools.partial(
        pl.kernel,
        out_type=jax.ShapeDtypeStruct((n, d), jnp.float32),
        mesh=mesh,
        compiler_params=pltpu.CompilerParams(use_tc_tiling_on_sc=False),
        scratch_types=[
            pltpu.VMEM((k, _G), jnp.int32),
            pltpu.VMEM((_CHUNK, 2 * d), jnp.float32),
            pltpu.SemaphoreType.DMA,
        ],
    )
    def gather_kernel(x_hbm, w_hbm, out_hbm, idx_v, rows_v, sem):
        wid = lax.axis_index("s") * _NC + lax.axis_index("c")
        base = wid * per_w
        pltpu.sync_copy(x_hbm.at[wid], idx_v)

        def chunk_body(c, carry):
            copies = []
            for g in range(_GPC):
                copies.append(
                    pltpu.async_copy(
                        w_hbm.at[idx_v.at[c * _GPC + g]],
                        rows_v.at[pl.ds(g * _G, _G)],
                        sem,
                    )
                )
            for cp in copies:
                cp.wait()
            pltpu.sync_copy(
                rows_v.at[:, pl.ds(0, d)],
                out_hbm.at[pl.ds(base + c * _CHUNK, _CHUNK)],
            )
            return carry

        lax.fori_loop(0, n_chunks, chunk_body, 0)

    return gather_kernel(x3, wpad)


def _y_embed_tc(y, w1):
    s, b = y.shape[1], y.shape[0]
    v, d = w1.shape
    rows = 2048
    nb = b // rows
    yt4 = y.T.reshape(s, nb, 1, rows)
    w1p = jnp.pad(w1, ((0, 16 - v), (0, 0)))

    def body(y_ref, w1_ref, o_ref):
        idx = y_ref[0, 0, 0, :]
        oh = (idx[:, None] == lax.broadcasted_iota(jnp.int32, (rows, 16), 1))
        o_ref[0] = jax.lax.dot(
            oh.astype(jnp.float32), w1_ref[...],
            precision=jax.lax.Precision.HIGHEST,
            preferred_element_type=jnp.float32,
        )

    out = pl.pallas_call(
        body,
        grid=(s, nb),
        in_specs=[
            pl.BlockSpec((1, 1, 1, rows), lambda i, j: (i, j, 0, 0)),
            pl.BlockSpec((16, d), lambda i, j: (0, 0)),
        ],
        out_specs=pl.BlockSpec((1, rows, d), lambda i, j: (i, j, 0)),
        out_shape=jax.ShapeDtypeStruct((s, b, d), jnp.float32),
    )(yt4, w1p)
    return out.transpose(1, 0, 2)


def kernel(x, w0, y, w1):
    b, s = x.shape
    n = b * s
    y_emb = _y_embed_tc(y.astype(jnp.int32), w1)
    wpad = jnp.pad(w0, ((0, 0), (0, 128 - w0.shape[1])))
    x_emb = _x_gather_sc(x.reshape(n).astype(jnp.int32), wpad, w0.shape[1])
    return (x_emb.reshape(b, s, w0.shape[1]), y_emb)


# speedup vs baseline: 1.8853x; 1.0579x over previous
import functools

import jax
import jax.numpy as jnp
from jax import lax
from jax.experimental import pallas as pl
from jax.experimental.pallas import tpu as pltpu
from jax.experimental.pallas import tpu_sc as plsc

_NC = 2
_NS = 16
_NW = _NC * _NS

_G = 128
_CHUNK = 256
_GPC = _CHUNK // _G


def _x_gather_sc(x_flat, wpad, d):
    n = x_flat.shape[0]
    per_w = n // _NW
    k = per_w // _G
    n_chunks = per_w // _CHUNK

    x3 = x_flat.reshape(_NW, k, _G)

    mesh = plsc.VectorSubcoreMesh(core_axis_name="c", subcore_axis_name="s")

    @functools.partial(
        pl.kernel,
        out_type=jax.ShapeDtypeStruct((n, d), jnp.float32),
        mesh=mesh,
        compiler_params=pltpu.CompilerParams(use_tc_tiling_on_sc=False),
        scratch_types=[
            pltpu.VMEM((k, _G), jnp.int32),
            pltpu.VMEM((_CHUNK, 2 * d), jnp.float32),
            pltpu.VMEM((_CHUNK, 2 * d), jnp.float32),
            pltpu.SemaphoreType.DMA,
            pltpu.SemaphoreType.DMA,
            pltpu.SemaphoreType.DMA,
            pltpu.SemaphoreType.DMA,
        ],
    )
    def gather_kernel(x_hbm, w_hbm, out_hbm, idx_v, rows0, rows1,
                      gs0, gs1, ws0, ws1):
        wid = lax.axis_index("s") * _NC + lax.axis_index("c")
        base = wid * per_w
        rows = (rows0, rows1)
        gs = (gs0, gs1)
        ws = (ws0, ws1)
        pltpu.sync_copy(x_hbm.at[wid], idx_v)

        def fire(c, bi):
            @pl.when(c < n_chunks)
            def _():
                for g in range(_GPC):
                    pltpu.async_copy(
                        w_hbm.at[idx_v.at[c * _GPC + g]],
                        rows[bi].at[pl.ds(g * _G, _G)],
                        gs[bi],
                    )

        fire(0, 0)

        def chunk_body(c2, carry):
            for b in range(2):
                c = c2 * 2 + b

                @pl.when(c >= 1)
                def _():
                    pltpu.make_async_copy(
                        out_hbm.at[pl.ds(0, _CHUNK)], rows[b ^ 1].at[:, pl.ds(0, d)],
                        ws[b ^ 1],
                    ).wait()

                fire(c + 1, b ^ 1)

                pltpu.make_async_copy(
                    w_hbm.at[pl.ds(0, _CHUNK)], rows[b], gs[b]
                ).wait()
                pltpu.async_copy(
                    rows[b].at[:, pl.ds(0, d)],
                    out_hbm.at[pl.ds(base + c * _CHUNK, _CHUNK)],
                    ws[b],
                )
            return carry

        lax.fori_loop(0, n_chunks // 2, chunk_body, 0)

        lb = (n_chunks - 1) % 2
        pltpu.make_async_copy(
            out_hbm.at[pl.ds(0, _CHUNK)], rows[lb].at[:, pl.ds(0, d)], ws[lb]
        ).wait()

    return gather_kernel(x3, wpad)


def _y_embed_tc(y, w1):
    s, b = y.shape[1], y.shape[0]
    v, d = w1.shape
    rows = 2048
    nb = b // rows
    yt4 = y.T.reshape(s, nb, 1, rows)
    w1p = jnp.pad(w1, ((0, 16 - v), (0, 0)))

    def body(y_ref, w1_ref, o_ref):
        idx = y_ref[0, 0, 0, :]
        oh = (idx[:, None] == lax.broadcasted_iota(jnp.int32, (rows, 16), 1))
        o_ref[0] = jax.lax.dot(
            oh.astype(jnp.float32), w1_ref[...],
            precision=jax.lax.Precision.HIGHEST,
            preferred_element_type=jnp.float32,
        )

    out = pl.pallas_call(
        body,
        grid=(s, nb),
        in_specs=[
            pl.BlockSpec((1, 1, 1, rows), lambda i, j: (i, j, 0, 0)),
            pl.BlockSpec((16, d), lambda i, j: (0, 0)),
        ],
        out_specs=pl.BlockSpec((1, rows, d), lambda i, j: (i, j, 0)),
        out_shape=jax.ShapeDtypeStruct((s, b, d), jnp.float32),
    )(yt4, w1p)
    return out.transpose(1, 0, 2)


def kernel(x, w0, y, w1):
    b, s = x.shape
    n = b * s
    y_emb = _y_embed_tc(y.astype(jnp.int32), w1)
    wpad = jnp.pad(w0, ((0, 0), (0, 128 - w0.shape[1])))
    xs = x.T.astype(jnp.int32).reshape(n)
    out = _x_gather_sc(xs, wpad, w0.shape[1])
    x_emb = out.reshape(s, b, w0.shape[1]).transpose(1, 0, 2)
    return (x_emb, y_emb)


# speedup vs baseline: 2.0493x; 1.0870x over previous
import functools

import jax
import jax.numpy as jnp
from jax import lax
from jax.experimental import pallas as pl
from jax.experimental.pallas import tpu as pltpu
from jax.experimental.pallas import tpu_sc as plsc

_NC = 2
_NS = 16
_NW = _NC * _NS

_G = 128
_CHUNK = 256
_GPC = _CHUNK // _G


def _x_gather_sc(x_flat, wpad, d):
    n = x_flat.shape[0]
    per_w = n // _NW
    k = per_w // _G
    n_chunks = per_w // _CHUNK

    x3 = x_flat.reshape(_NW, k, _G)

    mesh = plsc.VectorSubcoreMesh(core_axis_name="c", subcore_axis_name="s")

    @functools.partial(
        pl.kernel,
        out_type=jax.ShapeDtypeStruct((n, d), jnp.float32),
        mesh=mesh,
        compiler_params=pltpu.CompilerParams(use_tc_tiling_on_sc=False),
        scratch_types=[
            pltpu.VMEM((k, _G), jnp.int32),
            pltpu.VMEM((_CHUNK, 2 * d), jnp.float32),
            pltpu.VMEM((_CHUNK, 2 * d), jnp.float32),
            pltpu.SemaphoreType.DMA,
            pltpu.SemaphoreType.DMA,
            pltpu.SemaphoreType.DMA,
            pltpu.SemaphoreType.DMA,
        ],
    )
    def gather_kernel(x_hbm, w_hbm, out_hbm, idx_v, rows0, rows1,
                      gs0, gs1, ws0, ws1):
        wid = lax.axis_index("s") * _NC + lax.axis_index("c")
        base = wid * per_w
        rows = (rows0, rows1)
        gs = (gs0, gs1)
        ws = (ws0, ws1)
        pltpu.sync_copy(x_hbm.at[wid], idx_v)

        def fire(c, bi):
            @pl.when(c < n_chunks)
            def _():
                for g in range(_GPC):
                    pltpu.async_copy(
                        w_hbm.at[idx_v.at[c * _GPC + g]],
                        rows[bi].at[pl.ds(g * _G, _G)],
                        gs[bi],
                    )

        fire(0, 0)

        def chunk_body(c2, carry):
            for b in range(2):
                c = c2 * 2 + b

                @pl.when(c >= 1)
                def _():
                    pltpu.make_async_copy(
                        out_hbm.at[pl.ds(0, _CHUNK)], rows[b ^ 1].at[:, pl.ds(0, d)],
                        ws[b ^ 1],
                    ).wait()

                fire(c + 1, b ^ 1)

                pltpu.make_async_copy(
                    w_hbm.at[pl.ds(0, _CHUNK)], rows[b], gs[b]
                ).wait()
                pltpu.async_copy(
                    rows[b].at[:, pl.ds(0, d)],
                    out_hbm.at[pl.ds(base + c * _CHUNK, _CHUNK)],
                    ws[b],
                )
            return carry

        lax.fori_loop(0, n_chunks // 2, chunk_body, 0)

        lb = (n_chunks - 1) % 2
        pltpu.make_async_copy(
            out_hbm.at[pl.ds(0, _CHUNK)], rows[lb].at[:, pl.ds(0, d)], ws[lb]
        ).wait()

    return gather_kernel(x3, wpad)


def _y_embed_tc(y, w1):
    s, b = y.shape[1], y.shape[0]
    v, d = w1.shape
    rows = 2048
    nb = b // rows
    yt4 = y.T.reshape(s, nb, 1, rows)
    w1p = jnp.pad(w1, ((0, 16 - v), (0, 0)))

    def body(y_ref, w1_ref, o_ref):
        idx = y_ref[0, 0, 0, :]
        oh = (idx[:, None] == lax.broadcasted_iota(jnp.int32, (rows, 16), 1))
        o_ref[0] = jax.lax.dot(
            oh.astype(jnp.float32), w1_ref[...],
            precision=jax.lax.Precision.HIGHEST,
            preferred_element_type=jnp.float32,
        )

    out = pl.pallas_call(
        body,
        grid=(s, nb),
        in_specs=[
            pl.BlockSpec((1, 1, 1, rows), lambda i, j: (i, j, 0, 0)),
            pl.BlockSpec((16, d), lambda i, j: (0, 0)),
        ],
        out_specs=pl.BlockSpec((1, rows, d), lambda i, j: (i, j, 0)),
        out_shape=jax.ShapeDtypeStruct((s, b, d), jnp.float32),
    )(yt4, w1p)
    return out.transpose(1, 0, 2)


def kernel(x, w0, y, w1):
    b, s = x.shape
    n = b * s
    y_emb = _y_embed_tc(y.astype(jnp.int32), w1)
    wpad = jnp.pad(w0, ((0, 0), (0, 128 - w0.shape[1])))
    guard = (y_emb[0, 0, 0] * 0.0).astype(jnp.int32)
    xs = x.T.astype(jnp.int32).reshape(n) + guard
    out = _x_gather_sc(xs, wpad, w0.shape[1])
    x_emb = out.reshape(s, b, w0.shape[1]).transpose(1, 0, 2)
    return (x_emb, y_emb)


# speedup vs baseline: 2.0581x; 1.0043x over previous
import functools

import jax
import jax.numpy as jnp
from jax import lax
from jax.experimental import pallas as pl
from jax.experimental.pallas import tpu as pltpu
from jax.experimental.pallas import tpu_sc as plsc

_NC = 2
_NS = 16
_NW = _NC * _NS

_G = 128
_CHUNK = 256
_GPC = _CHUNK // _G


def _x_gather_sc(x_flat, wpad, d):
    n = x_flat.shape[0]
    per_w = n // _NW
    k = per_w // _G
    n_chunks = per_w // _CHUNK
    ns, nb = 26, n // 26

    x3 = x_flat.reshape(_NW, k, _G)

    mesh = plsc.VectorSubcoreMesh(core_axis_name="c", subcore_axis_name="s")

    @functools.partial(
        pl.kernel,
        out_type=jax.ShapeDtypeStruct((ns, nb, d), jnp.float32),
        mesh=mesh,
        compiler_params=pltpu.CompilerParams(use_tc_tiling_on_sc=False),
        scratch_types=[
            pltpu.VMEM((k, _G), jnp.int32),
            pltpu.VMEM((_CHUNK, 2 * d), jnp.float32),
            pltpu.VMEM((_CHUNK, 2 * d), jnp.float32),
            pltpu.SemaphoreType.DMA,
            pltpu.SemaphoreType.DMA,
            pltpu.SemaphoreType.DMA,
            pltpu.SemaphoreType.DMA,
        ],
    )
    def gather_kernel(x_hbm, w_hbm, out_hbm, idx_v, rows0, rows1,
                      gs0, gs1, ws0, ws1):
        wid = lax.axis_index("s") * _NC + lax.axis_index("c")
        base = wid * per_w
        rows = (rows0, rows1)
        gs = (gs0, gs1)
        ws = (ws0, ws1)
        pltpu.sync_copy(x_hbm.at[wid], idx_v)

        def fire(c, bi):
            @pl.when(c < n_chunks)
            def _():
                for g in range(_GPC):
                    pltpu.async_copy(
                        w_hbm.at[idx_v.at[c * _GPC + g]],
                        rows[bi].at[pl.ds(g * _G, _G)],
                        gs[bi],
                    )

        fire(0, 0)

        def chunk_body(c2, carry):
            for b in range(2):
                c = c2 * 2 + b

                @pl.when(c >= 1)
                def _():
                    pltpu.make_async_copy(
                        out_hbm.at[0, pl.ds(0, _CHUNK)],
                        rows[b ^ 1].at[:, pl.ds(0, d)],
                        ws[b ^ 1],
                    ).wait()

                fire(c + 1, b ^ 1)

                pltpu.make_async_copy(
                    w_hbm.at[pl.ds(0, _CHUNK)], rows[b], gs[b]
                ).wait()
                flat = base + c * _CHUNK
                pltpu.async_copy(
                    rows[b].at[:, pl.ds(0, d)],
                    out_hbm.at[flat // nb, pl.ds(flat % nb, _CHUNK)],
                    ws[b],
                )
            return carry

        lax.fori_loop(0, n_chunks // 2, chunk_body, 0)

        lb = (n_chunks - 1) % 2
        pltpu.make_async_copy(
            out_hbm.at[0, pl.ds(0, _CHUNK)], rows[lb].at[:, pl.ds(0, d)], ws[lb]
        ).wait()

    return gather_kernel(x3, wpad)


def _y_embed_tc(y, w1):
    s, b = y.shape[1], y.shape[0]
    v, d = w1.shape
    rows = 2048
    nb = b // rows
    yt4 = y.T.reshape(s, nb, 1, rows)
    w1p = jnp.pad(w1, ((0, 16 - v), (0, 0)))

    def body(y_ref, w1_ref, o_ref):
        idx = y_ref[0, 0, 0, :]
        oh = (idx[:, None] == lax.broadcasted_iota(jnp.int32, (rows, 16), 1))
        o_ref[0] = jax.lax.dot(
            oh.astype(jnp.float32), w1_ref[...],
            precision=jax.lax.Precision.HIGHEST,
            preferred_element_type=jnp.float32,
        )

    out = pl.pallas_call(
        body,
        grid=(s, nb),
        in_specs=[
            pl.BlockSpec((1, 1, 1, rows), lambda i, j: (i, j, 0, 0)),
            pl.BlockSpec((16, d), lambda i, j: (0, 0)),
        ],
        out_specs=pl.BlockSpec((1, rows, d), lambda i, j: (i, j, 0)),
        out_shape=jax.ShapeDtypeStruct((s, b, d), jnp.float32),
    )(yt4, w1p)
    return out.transpose(1, 0, 2)


def kernel(x, w0, y, w1):
    b, s = x.shape
    n = b * s
    y_emb = _y_embed_tc(y.astype(jnp.int32), w1)
    wpad = jnp.pad(w0, ((0, 0), (0, 128 - w0.shape[1])))
    guard = (y_emb[0, 0, 0] * 0.0).astype(jnp.int32)
    xs = x.T.astype(jnp.int32).reshape(n) + guard
    out = _x_gather_sc(xs, wpad, w0.shape[1])
    x_emb = out.transpose(1, 0, 2)
    return (x_emb, y_emb)
